# baseline (speedup 1.0000x reference)
import jax
import jax.numpy as jnp
from jax.experimental import pallas as pl
from jax.experimental.pallas import tpu as pltpu

_B = 16384
_C = 1000
_W = 4096
_NBLK = _B // _W
_K_RANK = int(_B * 0.3) + 1


def _ce_select_kernel(x_ref, t_ref, o_ref, ce_ref):
    i = pl.program_id(0)
    x = x_ref[...]
    t = t_ref[0]
    m = jnp.max(x, axis=0, keepdims=True)
    s = jnp.sum(jnp.exp(x - m), axis=0, keepdims=True)
    row = jax.lax.broadcasted_iota(jnp.int32, (_C, _W), 0)
    g = jnp.sum(jnp.where(row == t, x, 0.0), axis=0, keepdims=True)
    ce = m + jnp.log(s) - g
    ce_ref[pl.ds(i, 1), :] = ce

    @pl.when(i == _NBLK - 1)
    def _():
        ce_all = ce_ref[...]
        raw = jax.lax.bitcast_convert_type(ce_all, jnp.int32)
        keys = raw ^ ((raw >> 31) & jnp.int32(0x7FFFFFFF))
        nonneg = jnp.sum((keys >= 0).astype(jnp.int32))
        base0 = jnp.where(nonneg >= _K_RANK, jnp.int32(0),
                          jnp.int32(-2147483648))

        def body(b, base):
            cand = base + (jnp.int32(1) << (30 - b))
            cnt = jnp.sum((keys >= cand).astype(jnp.int32))
            return jnp.where(cnt >= _K_RANK, cand, base)

        lam = jax.lax.fori_loop(0, 31, body, base0)
        kept = jnp.where(keys >= lam, ce_all, 0.0)
        o_ref[0, 0] = jnp.sum(kept) / _B


def kernel(output, target):
    xt = output.T
    t3 = target.astype(jnp.int32).reshape(_NBLK, 1, _W)
    out = pl.pallas_call(
        _ce_select_kernel,
        grid=(_NBLK,),
        in_specs=[
            pl.BlockSpec((_C, _W), lambda i: (0, i)),
            pl.BlockSpec((1, 1, _W), lambda i: (i, 0, 0)),
        ],
        out_specs=pl.BlockSpec(memory_space=pltpu.SMEM),
        out_shape=jax.ShapeDtypeStruct((1, 1), jnp.float32),
        scratch_shapes=[pltpu.VMEM((_NBLK, _W), jnp.float32)],
    )(xt, t3)
    return out[0, 0]


# speedup vs baseline: 1.0387x; 1.0387x over previous
import jax
import jax.numpy as jnp
from jax.experimental import pallas as pl
from jax.experimental.pallas import tpu as pltpu

_B = 16384
_C = 1000
_W = 2048
_NBLK = _B // _W
_K_RANK = int(_B * 0.3) + 1


def _ce_select_kernel(x_ref, t_ref, o_ref, ce_ref):
    i = pl.program_id(0)
    x = x_ref[...]
    t = t_ref[0]
    m = jnp.max(x, axis=0, keepdims=True)
    s = jnp.sum(jnp.exp(x - m), axis=0, keepdims=True)
    row = jax.lax.broadcasted_iota(jnp.int32, (_C, _W), 0)
    g = jnp.sum(jnp.where(row == t, x, 0.0), axis=0, keepdims=True)
    ce = m + jnp.log(s) - g
    ce_ref[pl.ds(i, 1), :] = ce

    @pl.when(i == _NBLK - 1)
    def _():
        ce_all = ce_ref[...]
        raw = jax.lax.bitcast_convert_type(ce_all, jnp.int32)
        keys = raw ^ ((raw >> 31) & jnp.int32(0x7FFFFFFF))
        nonneg = jnp.sum((keys >= 0).astype(jnp.int32))
        base0 = jnp.where(nonneg >= _K_RANK, jnp.int32(0),
                          jnp.int32(-2147483648))

        def body(b, base):
            cand = base + (jnp.int32(1) << (30 - b))
            cnt = jnp.sum((keys >= cand).astype(jnp.int32))
            return jnp.where(cnt >= _K_RANK, cand, base)

        lam = jax.lax.fori_loop(0, 31, body, base0)
        kept = jnp.where(keys >= lam, ce_all, 0.0)
        o_ref[0, 0] = jnp.sum(kept) / _B


def kernel(output, target):
    xt = output.T
    t3 = target.astype(jnp.int32).reshape(_NBLK, 1, _W)
    out = pl.pallas_call(
        _ce_select_kernel,
        grid=(_NBLK,),
        in_specs=[
            pl.BlockSpec((_C, _W), lambda i: (0, i)),
            pl.BlockSpec((1, 1, _W), lambda i: (i, 0, 0)),
        ],
        out_specs=pl.BlockSpec(memory_space=pltpu.SMEM),
        out_shape=jax.ShapeDtypeStruct((1, 1), jnp.float32),
        scratch_shapes=[pltpu.VMEM((_NBLK, _W), jnp.float32)],
    )(xt, t3)
    return out[0, 0]
